# baseline (speedup 1.0000x reference)
import math

import jax
import jax.numpy as jnp
from jax.experimental import pallas as pl
import jax.experimental.pallas.tpu as pltpu
from jax.experimental.pallas import tpu_sc as plsc

N_HEADS = 16
N_KV_HEADS = 4
HEAD_SIZE = 128
MAX_SEQLEN = 2048
SPARSE_BLOCK = 64
VLLM_BLOCK = 16
LOCAL_BLOCKS = 8
VERT_STRIDE = 4
NUM_SEQS = 32
PAGES_PER_SPARSE = SPARSE_BLOCK // VLLM_BLOCK
NUM_SPARSE_BLOCKS = MAX_SEQLEN // SPARSE_BLOCK
MAX_ACTIVE = LOCAL_BLOCKS + (NUM_SPARSE_BLOCKS - LOCAL_BLOCKS) // VERT_STRIDE
PAGES_PER_SEQ = MAX_ACTIVE * PAGES_PER_SPARSE
T_ACT = MAX_ACTIVE * SPARSE_BLOCK

R = N_HEADS // N_KV_HEADS
SM_SCALE = 1.0 / math.sqrt(HEAD_SIZE)
NEG_INF = -1e30


NUM_TECS = 32
PAGES_TOTAL = NUM_SEQS * PAGES_PER_SEQ
PAGES_PER_TEC = PAGES_TOTAL // NUM_TECS


def _sc_gather_body(pages_hbm, k_hbm, v_hbm, kact_hbm, vact_hbm,
                    pages_vmem, pbuf, obuf, in_sem, out_sem):
    core = jax.lax.axis_index("c")
    sub = jax.lax.axis_index("s")
    tec = core * 16 + sub

    pltpu.make_async_copy(pages_hbm, pages_vmem.at[pl.ds(0, PAGES_TOTAL)], in_sem).start()
    pltpu.make_async_copy(pages_hbm, pages_vmem.at[pl.ds(0, PAGES_TOTAL)], in_sem).wait()

    lane = jax.lax.iota(jnp.int32, 16)

    def _one(i, src_hbm, dst_hbm):
        row = tec + i * NUM_TECS
        page = pages_vmem[pl.ds(row, 16)][0]
        pltpu.make_async_copy(src_hbm.at[page], pbuf, in_sem).start()
        pltpu.make_async_copy(src_hbm.at[page], pbuf, in_sem).wait()
        for h in range(N_KV_HEADS):
            for t in range(VLLM_BLOCK):
                for dg in range(HEAD_SIZE // 16):
                    vec = plsc.load_gather(
                        pbuf, [lane * 0 + h, dg * 16 + lane,
                               lane * 0 + t])
                    obuf[h, t, pl.ds(dg * 16, 16)] = vec
        pltpu.make_async_copy(obuf, dst_hbm.at[row], out_sem).start()
        pltpu.make_async_copy(obuf, dst_hbm.at[row], out_sem).wait()

    def _loop(i, carry):
        _one(i, k_hbm, kact_hbm)
        _one(i, v_hbm, vact_hbm)
        return carry

    jax.lax.fori_loop(0, PAGES_PER_TEC, _loop, 0)


def _sc_gather(pages, k, v):
    out_t = (jax.ShapeDtypeStruct((PAGES_TOTAL, N_KV_HEADS, VLLM_BLOCK,
                                   HEAD_SIZE), jnp.float32),
             jax.ShapeDtypeStruct((PAGES_TOTAL, N_KV_HEADS, VLLM_BLOCK,
                                   HEAD_SIZE), jnp.float32))
    return pl.kernel(
        _sc_gather_body,
        out_type=out_t,
        mesh=plsc.VectorSubcoreMesh(core_axis_name="c",
                                    subcore_axis_name="s"),
        compiler_params=pltpu.CompilerParams(needs_layout_passes=False),
        scratch_types=(
            pltpu.VMEM((PAGES_TOTAL + 16,), jnp.int32),
            pltpu.VMEM((N_KV_HEADS, HEAD_SIZE, VLLM_BLOCK), jnp.float32),
            pltpu.VMEM((N_KV_HEADS, VLLM_BLOCK, HEAD_SIZE), jnp.float32),
            pltpu.SemaphoreType.DMA,
            pltpu.SemaphoreType.DMA,
        ),
    )(pages, k, v)


def _attn_kernel(ctx_ref, q_ref, tok_ref, k_hbm, v_hbm, out_ref,
                 kbuf, vbuf, ksem, vsem):
    s = pl.program_id(0)
    slot = jax.lax.rem(s, 2)

    def _start(seq, slot):
        base = seq * PAGES_PER_SEQ
        pltpu.make_async_copy(k_hbm.at[pl.ds(base, PAGES_PER_SEQ)],
                              kbuf.at[slot], ksem.at[slot]).start()
        pltpu.make_async_copy(v_hbm.at[pl.ds(base, PAGES_PER_SEQ)],
                              vbuf.at[slot], vsem.at[slot]).start()

    @pl.when(s == 0)
    def _prologue():
        _start(s, slot)

    @pl.when(s + 1 < NUM_SEQS)
    def _prefetch_next():
        _start(s + 1, 1 - slot)

    pltpu.make_async_copy(k_hbm.at[pl.ds(0, PAGES_PER_SEQ)],
                          kbuf.at[slot], ksem.at[slot]).wait()
    pltpu.make_async_copy(v_hbm.at[pl.ds(0, PAGES_PER_SEQ)],
                          vbuf.at[slot], vsem.at[slot]).wait()

    valid = tok_ref[0, 0] < ctx_ref[s]
    outs = []
    for h in range(N_KV_HEADS):
        kh = kbuf[slot, :, h].reshape(T_ACT, HEAD_SIZE)
        vh = vbuf[slot, :, h].reshape(T_ACT, HEAD_SIZE)
        qh = q_ref[0, h]
        sc = jax.lax.dot_general(
            qh, kh,
            dimension_numbers=(((1,), (1,)), ((), ())),
            preferred_element_type=jnp.float32)
        sc = jnp.where(valid[None, :], sc, NEG_INF)
        m = jnp.max(sc, axis=1, keepdims=True)
        p = jnp.exp(sc - m)
        l = jnp.sum(p, axis=1, keepdims=True)
        pv = jax.lax.dot_general(
            p, vh,
            dimension_numbers=(((1,), (0,)), ((), ())),
            preferred_element_type=jnp.float32)
        outs.append(pv / l)
    out_ref[0] = jnp.concatenate(outs, axis=0)


@jax.jit
def kernel(q, k, v, block_tables, context_lens):
    qb = (context_lens - 1) // SPARSE_BLOCK
    n_str = jnp.maximum((qb - (LOCAL_BLOCKS - 1)) // VERT_STRIDE, 0)
    local_start = jnp.maximum(qb - (LOCAL_BLOCKS - 1), 0)
    n_act = n_str + jnp.minimum(qb + 1, LOCAL_BLOCKS)
    jj = jnp.arange(MAX_ACTIVE, dtype=jnp.int32)[None, :]
    blk = jnp.where(jj < n_str[:, None],
                    VERT_STRIDE * jj + (VERT_STRIDE - 1),
                    local_start[:, None] + (jj - n_str[:, None]))
    blk = jnp.minimum(blk, qb[:, None]).astype(jnp.int32)

    tok = (blk[:, :, None] * SPARSE_BLOCK
           + jnp.arange(SPARSE_BLOCK, dtype=jnp.int32))
    tok = jnp.where((jj < n_act[:, None])[:, :, None], tok, jnp.int32(1 << 30))
    tok = tok.reshape(NUM_SEQS, 1, T_ACT)

    first_page = jnp.take_along_axis(block_tables, blk * PAGES_PER_SPARSE,
                                     axis=1)
    first_page = jnp.where(jj < n_act[:, None], first_page,
                           block_tables[:, :1])
    pages = (first_page[..., None]
             + jnp.arange(PAGES_PER_SPARSE, dtype=jnp.int32))
    pages = pages.reshape(-1)
    kact, vact = _sc_gather(pages, k, v)

    q3 = (q * SM_SCALE).reshape(NUM_SEQS, N_KV_HEADS, R, HEAD_SIZE)

    grid_spec = pltpu.PrefetchScalarGridSpec(
        num_scalar_prefetch=1,
        grid=(NUM_SEQS,),
        in_specs=[
            pl.BlockSpec((1, N_KV_HEADS, R, HEAD_SIZE),
                         lambda s, *_: (s, 0, 0, 0)),
            pl.BlockSpec((1, 1, T_ACT), lambda s, *_: (s, 0, 0)),
            pl.BlockSpec(memory_space=pl.ANY),
            pl.BlockSpec(memory_space=pl.ANY),
        ],
        out_specs=pl.BlockSpec((1, N_HEADS, HEAD_SIZE),
                               lambda s, *_: (s, 0, 0)),
        scratch_shapes=[
            pltpu.VMEM((2, PAGES_PER_SEQ, N_KV_HEADS, VLLM_BLOCK,
                        HEAD_SIZE), jnp.float32),
            pltpu.VMEM((2, PAGES_PER_SEQ, N_KV_HEADS, VLLM_BLOCK,
                        HEAD_SIZE), jnp.float32),
            pltpu.SemaphoreType.DMA((2,)),
            pltpu.SemaphoreType.DMA((2,)),
        ],
    )

    return pl.pallas_call(
        _attn_kernel,
        grid_spec=grid_spec,
        out_shape=jax.ShapeDtypeStruct((NUM_SEQS, N_HEADS, HEAD_SIZE),
                                       jnp.float32),
    )(context_lens, q3, tok, kact, vact)


# speedup vs baseline: 12.7868x; 12.7868x over previous
import math

import jax
import jax.numpy as jnp
from jax.experimental import pallas as pl
import jax.experimental.pallas.tpu as pltpu

N_HEADS = 16
N_KV_HEADS = 4
HEAD_SIZE = 128
MAX_SEQLEN = 2048
SPARSE_BLOCK = 64
VLLM_BLOCK = 16
LOCAL_BLOCKS = 8
VERT_STRIDE = 4
NUM_SEQS = 32
PAGES_PER_SPARSE = SPARSE_BLOCK // VLLM_BLOCK
NUM_SPARSE_BLOCKS = MAX_SEQLEN // SPARSE_BLOCK
MAX_ACTIVE = LOCAL_BLOCKS + (NUM_SPARSE_BLOCKS - LOCAL_BLOCKS) // VERT_STRIDE
PAGES_PER_SEQ = MAX_ACTIVE * PAGES_PER_SPARSE
T_ACT = MAX_ACTIVE * SPARSE_BLOCK

R = N_HEADS // N_KV_HEADS
SM_SCALE = 1.0 / math.sqrt(HEAD_SIZE)
NEG_INF = -1e30


def _attn_kernel(ctx_ref, q_ref, tok_ref, k_hbm, v_hbm, out_ref,
                 kbuf, vbuf, ksem, vsem):
    s = pl.program_id(0)
    slot = jax.lax.rem(s, 2)

    def _start(seq, slot):
        base = seq * PAGES_PER_SEQ
        pltpu.make_async_copy(k_hbm.at[pl.ds(base, PAGES_PER_SEQ)],
                              kbuf.at[slot], ksem.at[slot]).start()
        pltpu.make_async_copy(v_hbm.at[pl.ds(base, PAGES_PER_SEQ)],
                              vbuf.at[slot], vsem.at[slot]).start()

    @pl.when(s == 0)
    def _prologue():
        _start(s, slot)

    @pl.when(s + 1 < NUM_SEQS)
    def _prefetch_next():
        _start(s + 1, 1 - slot)

    pltpu.make_async_copy(k_hbm.at[pl.ds(0, PAGES_PER_SEQ)],
                          kbuf.at[slot], ksem.at[slot]).wait()
    pltpu.make_async_copy(v_hbm.at[pl.ds(0, PAGES_PER_SEQ)],
                          vbuf.at[slot], vsem.at[slot]).wait()

    valid = tok_ref[0, 0] < ctx_ref[s]
    outs = []
    for h in range(N_KV_HEADS):
        kh = kbuf[slot, :, h].reshape(T_ACT, HEAD_SIZE)
        vh = vbuf[slot, :, h].reshape(T_ACT, HEAD_SIZE)
        qh = q_ref[0, h].astype(jnp.bfloat16)
        sc = jax.lax.dot_general(
            qh, kh,
            dimension_numbers=(((1,), (1,)), ((), ())),
            preferred_element_type=jnp.float32)
        sc = jnp.where(valid[None, :], sc, NEG_INF)
        m = jnp.max(sc, axis=1, keepdims=True)
        p = jnp.exp(sc - m)
        l = jnp.sum(p, axis=1, keepdims=True)
        pv = jax.lax.dot_general(
            p.astype(jnp.bfloat16), vh,
            dimension_numbers=(((1,), (0,)), ((), ())),
            preferred_element_type=jnp.float32)
        outs.append(pv / l)
    out_ref[0] = jnp.concatenate(outs, axis=0)


@jax.jit
def kernel(q, k, v, block_tables, context_lens):
    qb = (context_lens - 1) // SPARSE_BLOCK
    n_str = jnp.maximum((qb - (LOCAL_BLOCKS - 1)) // VERT_STRIDE, 0)
    local_start = jnp.maximum(qb - (LOCAL_BLOCKS - 1), 0)
    n_act = n_str + jnp.minimum(qb + 1, LOCAL_BLOCKS)
    jj = jnp.arange(MAX_ACTIVE, dtype=jnp.int32)[None, :]
    blk = jnp.where(jj < n_str[:, None],
                    VERT_STRIDE * jj + (VERT_STRIDE - 1),
                    local_start[:, None] + (jj - n_str[:, None]))
    blk = jnp.minimum(blk, qb[:, None]).astype(jnp.int32)

    tok = (blk[:, :, None] * SPARSE_BLOCK
           + jnp.arange(SPARSE_BLOCK, dtype=jnp.int32))
    tok = jnp.where((jj < n_act[:, None])[:, :, None], tok, jnp.int32(1 << 30))
    tok = tok.reshape(NUM_SEQS, 1, T_ACT)

    first_page = jnp.take_along_axis(block_tables, blk * PAGES_PER_SPARSE,
                                     axis=1)
    first_page = jnp.where(jj < n_act[:, None], first_page,
                           block_tables[:, :1])
    pages = (first_page[..., None]
             + jnp.arange(PAGES_PER_SPARSE, dtype=jnp.int32))
    pages = pages.reshape(-1)
    kact = jnp.take(k, pages, axis=0).transpose(0, 1, 3, 2).astype(jnp.bfloat16)
    vact = jnp.take(v, pages, axis=0).transpose(0, 1, 3, 2).astype(jnp.bfloat16)

    q3 = (q * SM_SCALE).reshape(NUM_SEQS, N_KV_HEADS, R, HEAD_SIZE)

    grid_spec = pltpu.PrefetchScalarGridSpec(
        num_scalar_prefetch=1,
        grid=(NUM_SEQS,),
        in_specs=[
            pl.BlockSpec((1, N_KV_HEADS, R, HEAD_SIZE),
                         lambda s, *_: (s, 0, 0, 0)),
            pl.BlockSpec((1, 1, T_ACT), lambda s, *_: (s, 0, 0)),
            pl.BlockSpec(memory_space=pl.ANY),
            pl.BlockSpec(memory_space=pl.ANY),
        ],
        out_specs=pl.BlockSpec((1, N_HEADS, HEAD_SIZE),
                               lambda s, *_: (s, 0, 0)),
        scratch_shapes=[
            pltpu.VMEM((2, PAGES_PER_SEQ, N_KV_HEADS, VLLM_BLOCK,
                        HEAD_SIZE), jnp.bfloat16),
            pltpu.VMEM((2, PAGES_PER_SEQ, N_KV_HEADS, VLLM_BLOCK,
                        HEAD_SIZE), jnp.bfloat16),
            pltpu.SemaphoreType.DMA((2,)),
            pltpu.SemaphoreType.DMA((2,)),
        ],
    )

    return pl.pallas_call(
        _attn_kernel,
        grid_spec=grid_spec,
        out_shape=jax.ShapeDtypeStruct((NUM_SEQS, N_HEADS, HEAD_SIZE),
                                       jnp.float32),
    )(context_lens, q3, tok, kact, vact)
